# baseline (speedup 1.0000x reference)
import functools

import jax
import jax.numpy as jnp
from jax import lax
from jax.experimental import pallas as pl
from jax.experimental.pallas import tpu as pltpu
from jax.experimental.pallas import tpu_sc as plsc

NUM_ROWS = 1000000
DIM = 64
BATCH = 4096
SEQ = 200
TOTAL = BATCH * SEQ

NC = 2
NS = 16
NW = NC * NS
PER_W = TOTAL // NW
SUB = 128
K = 8
CHUNK = K * SUB
N_STEPS = PER_W // CHUNK

_mesh = plsc.VectorSubcoreMesh(core_axis_name="c", subcore_axis_name="s")


@functools.partial(
    pl.kernel,
    out_type=jax.ShapeDtypeStruct((TOTAL, DIM), jnp.float32),
    mesh=_mesh,
    scratch_types=[
        pltpu.VMEM((K, SUB), jnp.int32),
        pltpu.VMEM((CHUNK, DIM), jnp.float32),
        pltpu.SemaphoreType.DMA,
    ],
    compiler_params=pltpu.CompilerParams(use_tc_tiling_on_sc=False),
)
def _gather_kernel(idx_hbm, table_hbm, out_hbm, idx_v, rows_v, sem):
    wid = lax.axis_index("s") * NC + lax.axis_index("c")
    base = wid * PER_W

    @pl.loop(0, N_STEPS)
    def _step(c):
        row0 = pl.multiple_of(base + c * CHUNK, CHUNK)
        pltpu.sync_copy(idx_hbm.at[pl.ds(pl.multiple_of(row0 // SUB, K), K)], idx_v)
        copies = [
            pltpu.async_copy(
                table_hbm.at[idx_v.at[j]],
                rows_v.at[pl.ds(j * SUB, SUB)],
                sem,
            )
            for j in range(K)
        ]
        for cp in copies:
            cp.wait()
        pltpu.sync_copy(rows_v, out_hbm.at[pl.ds(row0, CHUNK)])


def kernel(input, table):
    idx2d = input.reshape(TOTAL // SUB, SUB)
    out = _gather_kernel(idx2d, table)
    return out.reshape(BATCH, SEQ, DIM)


# speedup vs baseline: 1.0063x; 1.0063x over previous
import functools

import jax
import jax.numpy as jnp
from jax import lax
from jax.experimental import pallas as pl
from jax.experimental.pallas import tpu as pltpu
from jax.experimental.pallas import tpu_sc as plsc

NUM_ROWS = 1000000
DIM = 64
BATCH = 4096
SEQ = 200
TOTAL = BATCH * SEQ

NC = 2
NS = 16
NW = NC * NS
PER_W = TOTAL // NW
SUB = 128
K = 4
CHUNK = K * SUB
N_STEPS = PER_W // CHUNK

_mesh = plsc.VectorSubcoreMesh(core_axis_name="c", subcore_axis_name="s")


@functools.partial(
    pl.kernel,
    out_type=jax.ShapeDtypeStruct((TOTAL, DIM), jnp.float32),
    mesh=_mesh,
    scratch_types=[
        pltpu.VMEM((CHUNK,), jnp.int32),
        pltpu.VMEM((CHUNK,), jnp.int32),
        pltpu.VMEM((CHUNK, DIM), jnp.float32),
        pltpu.VMEM((CHUNK, DIM), jnp.float32),
        pltpu.SemaphoreType.DMA,
        pltpu.SemaphoreType.DMA,
        pltpu.SemaphoreType.DMA,
        pltpu.SemaphoreType.DMA,
    ],
    compiler_params=pltpu.CompilerParams(use_tc_tiling_on_sc=False),
)
def _gather_kernel(
    idx_hbm, table_hbm, out_hbm, idx0, idx1, rows0, rows1, g0, g1, s0, s1
):
    wid = lax.axis_index("s") * NC + lax.axis_index("c")
    base = wid * PER_W
    idx_v = (idx0, idx1)
    rows_v = (rows0, rows1)
    gsem = (g0, g1)
    ssem = (s0, s1)

    def row0_of(i):
        return pl.multiple_of(base + i * CHUNK, CHUNK)

    def load_idx(i, b):
        pltpu.sync_copy(idx_hbm.at[pl.ds(row0_of(i), CHUNK)], idx_v[b])

    def fire_gathers(b):
        for j in range(K):
            pltpu.async_copy(
                table_hbm.at[idx_v[b].at[pl.ds(j * SUB, SUB)]],
                rows_v[b].at[pl.ds(j * SUB, SUB)],
                gsem[b],
            )

    def drain_gathers(b):
        pltpu.make_async_copy(
            table_hbm.at[pl.ds(0, CHUNK)], rows_v[b], gsem[b]
        ).wait()

    def fire_store(i, b):
        pltpu.async_copy(rows_v[b], out_hbm.at[pl.ds(row0_of(i), CHUNK)], ssem[b])

    def wait_store(b):
        pltpu.make_async_copy(
            rows_v[b], out_hbm.at[pl.ds(0, CHUNK)], ssem[b]
        ).wait()

    load_idx(0, 0)
    fire_gathers(0)
    load_idx(1, 1)
    fire_gathers(1)
    drain_gathers(0)
    fire_store(0, 0)

    @pl.loop(2, N_STEPS, step=2)
    def _step(c):
        for b in range(2):
            i = c + b
            wait_store(b)
            load_idx(i, b)
            fire_gathers(b)
            drain_gathers(1 - b)
            fire_store(i - 1, 1 - b)

    drain_gathers(1)
    fire_store(N_STEPS - 1, 1)
    wait_store(0)
    wait_store(1)


def kernel(input, table):
    out = _gather_kernel(input.reshape(TOTAL), table)
    return out.reshape(BATCH, SEQ, DIM)
